# baseline (device time: 27827 ns/iter reference)
import jax
import jax.numpy as jnp
from jax import lax
from jax.experimental import pallas as pl
from jax.experimental.pallas import tpu as pltpu

N_DEV = 4
T = 512
D = 1024
V_LOC = 8192
N_CHUNKS = 2
VC = V_LOC // N_CHUNKS


def kernel(x, W, labels):
    labels_col = labels.reshape(T, 1)

    def body(x_ref, w_hbm, lab_ref, out_ref, comm_ref, wbuf0, wbuf1,
             copy_sems, send_sems, recv_sems):
        my_pos = lax.axis_index("i")
        wbufs = [wbuf0, wbuf1]

        def copy_of(k):
            return pltpu.make_async_copy(
                w_hbm.at[:, pl.ds(k * VC, VC)],
                wbufs[k % 2],
                copy_sems.at[k % 2],
            )

        copy_of(0).start()

        xv = x_ref[:]
        lab_base = lab_ref[:] - my_pos * V_LOC
        col = lax.broadcasted_iota(jnp.int32, (T, VC), 1)
        ss, cs = [], []
        for k in range(N_CHUNKS):
            if k + 1 < N_CHUNKS:
                copy_of(k + 1).start()
            copy_of(k).wait()
            lg = jnp.dot(
                xv, wbufs[k % 2][:], preferred_element_type=jnp.float32
            )
            ss.append(jnp.sum(jnp.exp(lg), axis=1, keepdims=True))
            cs.append(jnp.sum(
                jnp.where(col == (lab_base - k * VC), lg, 0.0),
                axis=1, keepdims=True,
            ))

        s = sum(ss)
        c = sum(cs)

        chunk = jnp.concatenate(
            [
                s.reshape(1, T),
                c.reshape(1, T),
                jnp.zeros((6, T), jnp.float32),
            ],
            axis=0,
        )
        comm_ref[pl.ds(my_pos, 1)] = chunk[None]

        barrier_sem = pltpu.get_barrier_semaphore()
        for d in range(1, N_DEV):
            peer = (my_pos + d) % N_DEV
            pl.semaphore_signal(
                barrier_sem, inc=1,
                device_id=(peer,), device_id_type=pl.DeviceIdType.MESH,
            )
        pl.semaphore_wait(barrier_sem, N_DEV - 1)

        sends = []
        for d in range(1, N_DEV):
            tgt = (my_pos + d) % N_DEV
            rdma = pltpu.make_async_remote_copy(
                src_ref=comm_ref.at[my_pos],
                dst_ref=comm_ref.at[my_pos],
                send_sem=send_sems.at[d - 1],
                recv_sem=recv_sems.at[my_pos],
                device_id=(tgt,),
                device_id_type=pl.DeviceIdType.MESH,
            )
            rdma.start()
            sends.append(rdma)

        for d in range(1, N_DEV):
            src_dev = (my_pos - d) % N_DEV
            recv = pltpu.make_async_remote_copy(
                src_ref=comm_ref.at[my_pos],
                dst_ref=comm_ref.at[src_dev],
                send_sem=send_sems.at[d - 1],
                recv_sem=recv_sems.at[src_dev],
                device_id=(src_dev,),
                device_id_type=pl.DeviceIdType.MESH,
            )
            recv.wait_recv()

        stats = comm_ref[:]
        gsum = jnp.sum(stats[:, 0, :], axis=0, keepdims=True)
        glab = jnp.sum(stats[:, 1, :], axis=0, keepdims=True)
        out_ref[:] = jnp.log(gsum) - glab

        for rdma in sends:
            rdma.wait_send()

    out = pl.pallas_call(
        body,
        out_shape=jax.ShapeDtypeStruct((1, T), jnp.float32),
        in_specs=[
            pl.BlockSpec(memory_space=pltpu.VMEM),
            pl.BlockSpec(memory_space=pl.ANY),
            pl.BlockSpec(memory_space=pltpu.VMEM),
        ],
        out_specs=pl.BlockSpec(memory_space=pltpu.VMEM),
        scratch_shapes=[
            pltpu.VMEM((N_DEV, 8, T), jnp.float32),
            pltpu.VMEM((D, VC), jnp.float32),
            pltpu.VMEM((D, VC), jnp.float32),
            pltpu.SemaphoreType.DMA((2,)),
            pltpu.SemaphoreType.DMA((N_DEV - 1,)),
            pltpu.SemaphoreType.DMA((N_DEV,)),
        ],
        compiler_params=pltpu.CompilerParams(
            collective_id=0,
            vmem_limit_bytes=60 * 1024 * 1024,
        ),
    )(x, W, labels_col)
    return out.reshape(T)


# device time: 26344 ns/iter; 1.0563x vs baseline; 1.0563x over previous
import jax
import jax.numpy as jnp
from jax import lax
from jax.experimental import pallas as pl
from jax.experimental.pallas import tpu as pltpu

N_DEV = 4
T = 512
D = 1024
V_LOC = 8192
N_CHUNKS = 4
VC = V_LOC // N_CHUNKS


def kernel(x, W, labels):
    labels_col = labels.reshape(T, 1)

    def body(x_ref, w_hbm, lab_ref, out_ref, comm_ref, wbuf0, wbuf1,
             copy_sems, send_sems, recv_sems):
        my_pos = lax.axis_index("i")
        wbufs = [wbuf0, wbuf1]

        def copy_of(k):
            return pltpu.make_async_copy(
                w_hbm.at[:, pl.ds(k * VC, VC)],
                wbufs[k % 2],
                copy_sems.at[k % 2],
            )

        copy_of(0).start()

        xv = x_ref[:]
        lab_base = lab_ref[:] - my_pos * V_LOC
        col = lax.broadcasted_iota(jnp.int32, (T, VC), 1)
        ss, cs = [], []
        for k in range(N_CHUNKS):
            if k + 1 < N_CHUNKS:
                copy_of(k + 1).start()
            copy_of(k).wait()
            lg = jnp.dot(
                xv, wbufs[k % 2][:], preferred_element_type=jnp.float32
            )
            ss.append(jnp.sum(jnp.exp(lg), axis=1, keepdims=True))
            cs.append(jnp.sum(
                jnp.where(col == (lab_base - k * VC), lg, 0.0),
                axis=1, keepdims=True,
            ))

        s = sum(ss)
        c = sum(cs)

        chunk = jnp.concatenate(
            [
                s.reshape(1, T),
                c.reshape(1, T),
                jnp.zeros((6, T), jnp.float32),
            ],
            axis=0,
        )
        comm_ref[pl.ds(my_pos, 1)] = chunk[None]

        barrier_sem = pltpu.get_barrier_semaphore()
        for d in range(1, N_DEV):
            peer = (my_pos + d) % N_DEV
            pl.semaphore_signal(
                barrier_sem, inc=1,
                device_id=(peer,), device_id_type=pl.DeviceIdType.MESH,
            )
        pl.semaphore_wait(barrier_sem, N_DEV - 1)

        sends = []
        for d in range(1, N_DEV):
            tgt = (my_pos + d) % N_DEV
            rdma = pltpu.make_async_remote_copy(
                src_ref=comm_ref.at[my_pos],
                dst_ref=comm_ref.at[my_pos],
                send_sem=send_sems.at[d - 1],
                recv_sem=recv_sems.at[my_pos],
                device_id=(tgt,),
                device_id_type=pl.DeviceIdType.MESH,
            )
            rdma.start()
            sends.append(rdma)

        for d in range(1, N_DEV):
            src_dev = (my_pos - d) % N_DEV
            recv = pltpu.make_async_remote_copy(
                src_ref=comm_ref.at[my_pos],
                dst_ref=comm_ref.at[src_dev],
                send_sem=send_sems.at[d - 1],
                recv_sem=recv_sems.at[src_dev],
                device_id=(src_dev,),
                device_id_type=pl.DeviceIdType.MESH,
            )
            recv.wait_recv()

        stats = comm_ref[:]
        gsum = jnp.sum(stats[:, 0, :], axis=0, keepdims=True)
        glab = jnp.sum(stats[:, 1, :], axis=0, keepdims=True)
        out_ref[:] = jnp.log(gsum) - glab

        for rdma in sends:
            rdma.wait_send()

    out = pl.pallas_call(
        body,
        out_shape=jax.ShapeDtypeStruct((1, T), jnp.float32),
        in_specs=[
            pl.BlockSpec(memory_space=pltpu.VMEM),
            pl.BlockSpec(memory_space=pl.ANY),
            pl.BlockSpec(memory_space=pltpu.VMEM),
        ],
        out_specs=pl.BlockSpec(memory_space=pltpu.VMEM),
        scratch_shapes=[
            pltpu.VMEM((N_DEV, 8, T), jnp.float32),
            pltpu.VMEM((D, VC), jnp.float32),
            pltpu.VMEM((D, VC), jnp.float32),
            pltpu.SemaphoreType.DMA((2,)),
            pltpu.SemaphoreType.DMA((N_DEV - 1,)),
            pltpu.SemaphoreType.DMA((N_DEV,)),
        ],
        compiler_params=pltpu.CompilerParams(
            collective_id=0,
            vmem_limit_bytes=60 * 1024 * 1024,
        ),
    )(x, W, labels_col)
    return out.reshape(T)


# device time: 26156 ns/iter; 1.0639x vs baseline; 1.0072x over previous
import jax
import jax.numpy as jnp
from jax import lax
from jax.experimental import pallas as pl
from jax.experimental.pallas import tpu as pltpu

N_DEV = 4
T = 512
D = 1024
V_LOC = 8192
N_CHUNKS = 4
VC = V_LOC // N_CHUNKS


def kernel(x, W, labels):
    labels_col = labels.reshape(T, 1)

    def body(x_ref, w_hbm, lab_ref, out_ref, comm_ref, wbuf0, wbuf1,
             copy_sems, send_semsA, recv_semsA, send_semsB, recv_semsB):
        my_pos = lax.axis_index("i")
        wbufs = [wbuf0, wbuf1]

        def copy_of(k):
            return pltpu.make_async_copy(
                w_hbm.at[:, pl.ds(k * VC, VC)],
                wbufs[k % 2],
                copy_sems.at[k % 2],
            )

        copy_of(0).start()

        xv = x_ref[:]
        lab_base = lab_ref[:] - my_pos * V_LOC
        col = lax.broadcasted_iota(jnp.int32, (T, VC), 1)

        def chunk_stats(k):
            lg = jnp.dot(
                xv, wbufs[k % 2][:], preferred_element_type=jnp.float32
            )
            sk = jnp.sum(jnp.exp(lg), axis=1, keepdims=True)
            ck = jnp.sum(
                jnp.where(col == (lab_base - k * VC), lg, 0.0),
                axis=1, keepdims=True,
            )
            return sk, ck

        def pack(s_col, c_col):
            return jnp.concatenate(
                [s_col.reshape(1, T), c_col.reshape(1, T)], axis=0
            )

        def exchange(rows, send_sems, recv_sems, do_start):
            sends, recvs = [], []
            for d in range(1, N_DEV):
                tgt = (my_pos + d) % N_DEV
                rdma = pltpu.make_async_remote_copy(
                    src_ref=comm_ref.at[my_pos, pl.ds(rows, 2)],
                    dst_ref=comm_ref.at[my_pos, pl.ds(rows, 2)],
                    send_sem=send_sems.at[d - 1],
                    recv_sem=recv_sems.at[my_pos],
                    device_id=(tgt,),
                    device_id_type=pl.DeviceIdType.MESH,
                )
                if do_start:
                    rdma.start()
                sends.append(rdma)
                src_dev = (my_pos - d) % N_DEV
                recvs.append(pltpu.make_async_remote_copy(
                    src_ref=comm_ref.at[my_pos, pl.ds(rows, 2)],
                    dst_ref=comm_ref.at[src_dev, pl.ds(rows, 2)],
                    send_sem=send_sems.at[d - 1],
                    recv_sem=recv_sems.at[src_dev],
                    device_id=(src_dev,),
                    device_id_type=pl.DeviceIdType.MESH,
                ))
            return sends, recvs

        ss, cs = [], []
        for k in range(N_CHUNKS - 1):
            copy_of(k + 1).start()
            copy_of(k).wait()
            sk, ck = chunk_stats(k)
            ss.append(sk)
            cs.append(ck)

        comm_ref[pl.ds(my_pos, 1), 0:2, :] = pack(sum(ss), sum(cs))[None]

        barrier_sem = pltpu.get_barrier_semaphore()
        for d in range(1, N_DEV):
            peer = (my_pos + d) % N_DEV
            pl.semaphore_signal(
                barrier_sem, inc=1,
                device_id=(peer,), device_id_type=pl.DeviceIdType.MESH,
            )
        pl.semaphore_wait(barrier_sem, N_DEV - 1)

        sendsA, recvsA = exchange(0, send_semsA, recv_semsA, do_start=True)

        copy_of(N_CHUNKS - 1).wait()
        sB, cB = chunk_stats(N_CHUNKS - 1)
        comm_ref[pl.ds(my_pos, 1), 2:4, :] = pack(sB, cB)[None]

        sendsB, recvsB = exchange(2, send_semsB, recv_semsB, do_start=True)

        for r in recvsA:
            r.wait_recv()
        for r in recvsB:
            r.wait_recv()

        stats = comm_ref[:]
        gsum = jnp.sum(stats[:, 0, :] + stats[:, 2, :], axis=0,
                       keepdims=True)
        glab = jnp.sum(stats[:, 1, :] + stats[:, 3, :], axis=0,
                       keepdims=True)
        out_ref[:] = jnp.log(gsum) - glab

        for rdma in sendsA + sendsB:
            rdma.wait_send()

    out = pl.pallas_call(
        body,
        out_shape=jax.ShapeDtypeStruct((1, T), jnp.float32),
        in_specs=[
            pl.BlockSpec(memory_space=pltpu.VMEM),
            pl.BlockSpec(memory_space=pl.ANY),
            pl.BlockSpec(memory_space=pltpu.VMEM),
        ],
        out_specs=pl.BlockSpec(memory_space=pltpu.VMEM),
        scratch_shapes=[
            pltpu.VMEM((N_DEV, 8, T), jnp.float32),
            pltpu.VMEM((D, VC), jnp.float32),
            pltpu.VMEM((D, VC), jnp.float32),
            pltpu.SemaphoreType.DMA((2,)),
            pltpu.SemaphoreType.DMA((N_DEV - 1,)),
            pltpu.SemaphoreType.DMA((N_DEV,)),
            pltpu.SemaphoreType.DMA((N_DEV - 1,)),
            pltpu.SemaphoreType.DMA((N_DEV,)),
        ],
        compiler_params=pltpu.CompilerParams(
            collective_id=0,
            vmem_limit_bytes=60 * 1024 * 1024,
        ),
    )(x, W, labels_col)
    return out.reshape(T)
